# baseline (speedup 1.0000x reference)
import functools

import jax
import jax.numpy as jnp
from jax.experimental import pallas as pl

N_NODES = 10000
N_EDGES = 320000
ND = 128
ED = 128
ALPHA = 0.1
BE = 2000


def _leaky(v, a):
    return jnp.where(v > 0, v, a * v)


def _mm2_body(a_ref, w1_ref, w2_ref, o1_ref, o2_ref):
    a = a_ref[...]
    o1_ref[...] = jax.lax.dot(a, w1_ref[...], precision=jax.lax.Precision.HIGHEST)
    o2_ref[...] = jax.lax.dot(a, w2_ref[...], precision=jax.lax.Precision.HIGHEST)


def _mm2(a, w1, w2):
    E, K = a.shape
    grid = (E // BE,)
    return pl.pallas_call(
        _mm2_body,
        grid=grid,
        in_specs=[
            pl.BlockSpec((BE, K), lambda i: (i, 0)),
            pl.BlockSpec((K, w1.shape[1]), lambda i: (0, 0)),
            pl.BlockSpec((K, w2.shape[1]), lambda i: (0, 0)),
        ],
        out_specs=[
            pl.BlockSpec((BE, w1.shape[1]), lambda i: (i, 0)),
            pl.BlockSpec((BE, w2.shape[1]), lambda i: (i, 0)),
        ],
        out_shape=[
            jax.ShapeDtypeStruct((E, w1.shape[1]), jnp.float32),
            jax.ShapeDtypeStruct((E, w2.shape[1]), jnp.float32),
        ],
    )(a, w1, w2)


def _edge_out_body(m_ref, efb_ref, wm_ref, we_ref, ef_ref, conf_ref):
    raw = jax.lax.dot(m_ref[...], wm_ref[...], precision=jax.lax.Precision.HIGHEST)
    ef1 = _leaky(raw + efb_ref[...], ALPHA)
    ef2 = _leaky(ef1, 0.1)
    ef_ref[...] = ef2
    conf_ref[...] = jax.lax.dot(ef2, we_ref[...], precision=jax.lax.Precision.HIGHEST)


def _edge_out(m, efb, wm, we_pad, final):
    E = m.shape[0]
    grid = (E // BE,)
    return pl.pallas_call(
        _edge_out_body,
        grid=grid,
        in_specs=[
            pl.BlockSpec((BE, ND), lambda i: (i, 0)),
            pl.BlockSpec((BE, ED), lambda i: (i, 0)),
            pl.BlockSpec((ND, ED), lambda i: (0, 0)),
            pl.BlockSpec((ED, we_pad.shape[1]), lambda i: (0, 0)),
        ],
        out_specs=[
            pl.BlockSpec((BE, ED), lambda i: (i, 0)),
            pl.BlockSpec((BE, we_pad.shape[1]), lambda i: (i, 0)),
        ],
        out_shape=[
            jax.ShapeDtypeStruct((E, ED), jnp.float32),
            jax.ShapeDtypeStruct((E, we_pad.shape[1]), jnp.float32),
        ],
    )(m, efb, wm, we_pad)


def _node_out_body(z2_ref, x_ref, wa_ref, wb_ref, wn_ref, x_new_ref, conf_ref):
    ft = (jax.lax.dot(z2_ref[...], wa_ref[...], precision=jax.lax.Precision.HIGHEST)
          + jax.lax.dot(x_ref[...], wb_ref[...], precision=jax.lax.Precision.HIGHEST))
    xn = _leaky(_leaky(ft, ALPHA), 0.1)
    x_new_ref[...] = xn
    conf_ref[...] = jax.lax.dot(xn, wn_ref[...], precision=jax.lax.Precision.HIGHEST)


def _node_out(z2, x, wa, wb, wn_pad):
    N = x.shape[0]
    return pl.pallas_call(
        _node_out_body,
        grid=(1,),
        in_specs=[
            pl.BlockSpec((N, ED), lambda i: (0, 0)),
            pl.BlockSpec((N, ND), lambda i: (0, 0)),
            pl.BlockSpec((ED, ND), lambda i: (0, 0)),
            pl.BlockSpec((ND, ND), lambda i: (0, 0)),
            pl.BlockSpec((ND, wn_pad.shape[1]), lambda i: (0, 0)),
        ],
        out_specs=[
            pl.BlockSpec((N, ND), lambda i: (0, 0)),
            pl.BlockSpec((N, wn_pad.shape[1]), lambda i: (0, 0)),
        ],
        out_shape=[
            jax.ShapeDtypeStruct((N, ND), jnp.float32),
            jax.ShapeDtypeStruct((N, wn_pad.shape[1]), jnp.float32),
        ],
    )(z2, x, wa, wb, wn_pad)


def _mm1_body(a_ref, w_ref, o_ref):
    o_ref[...] = jax.lax.dot(a_ref[...], w_ref[...],
                             precision=jax.lax.Precision.HIGHEST)


def _mm1(a, w):
    M, K = a.shape
    return pl.pallas_call(
        _mm1_body,
        grid=(1,),
        in_specs=[pl.BlockSpec((M, K), lambda i: (0, 0)),
                  pl.BlockSpec((K, w.shape[1]), lambda i: (0, 0))],
        out_specs=pl.BlockSpec((M, w.shape[1]), lambda i: (0, 0)),
        out_shape=jax.ShapeDtypeStruct((M, w.shape[1]), jnp.float32),
    )(a, w)


def _layer(x, ef, src, dst, fa, fnup, feup, wn_pad, we_pad, final):
    faE, faX = fa[:ED], fa[ED:]
    fnupA, fnupB = fnup[:ED], fnup[ED:]
    feupM, feupE = feup[:ND], feup[ND:]

    efa, efb = _mm2(ef, faE, feupE)
    xa = _mm1(x, faX)

    gp = xa[dst]
    gr = xa[src]
    raw_p = efa + gp
    raw_r = efa + gr
    a_p = jnp.exp(raw_p - jnp.max(raw_p, axis=0))
    a_r = jnp.exp(raw_r - jnp.max(raw_r, axis=0))

    z = jax.ops.segment_sum(a_p, dst, num_segments=N_NODES)
    a = a_p / (z[dst] + 1e-05)
    z2 = jax.ops.segment_sum(a * ef, dst, num_segments=N_NODES)

    s = a_p + a_r
    m = (a_r / s) * x[src] + (a_p / s) * x[dst]

    x_new, node_conf = _node_out(z2, x, fnupA, fnupB, wn_pad)
    ef_new, edge_conf = _edge_out(m, efb, feupM, we_pad, final)
    return x_new, ef_new, node_conf, edge_conf


def kernel(x, edge_feats, fa0, fnup0, feup0, fa1, fnup1, feup1, Wn, We, edge_index):
    src = edge_index[0]
    dst = edge_index[1]
    wn_pad = jnp.pad(Wn, ((0, 0), (0, 64 - Wn.shape[1])))
    we_pad = jnp.pad(We, ((0, 0), (0, 8 - We.shape[1])))

    nf, ef = x, edge_feats
    nf, ef, _, _ = _layer(nf, ef, src, dst, fa0, fnup0, feup0, wn_pad, we_pad, False)
    nf, ef, node_conf, edge_conf = _layer(nf, ef, src, dst, fa1, fnup1, feup1,
                                          wn_pad, we_pad, True)
    return (nf, ef, node_conf[:, :Wn.shape[1]], edge_conf[:, :We.shape[1]])


# speedup vs baseline: 1.7601x; 1.7601x over previous
import functools

import jax
import jax.numpy as jnp
from jax import lax
from jax.experimental import pallas as pl
from jax.experimental.pallas import tpu as pltpu
from jax.experimental.pallas import tpu_sc as plsc

N_NODES = 10000
N_EDGES = 320000
ND = 128
ED = 128
ALPHA = 0.1
BE = 2000
CH = 80
SR = 80

_MESH = plsc.VectorSubcoreMesh(core_axis_name="c", subcore_axis_name="s")


def _leaky(v, a):
    return jnp.where(v > 0, v, a * v)


def _mm2_body(a_ref, w1_ref, w2_ref, o1_ref, o2_ref):
    a = a_ref[...]
    o1_ref[...] = jax.lax.dot(a, w1_ref[...], precision=jax.lax.Precision.HIGHEST)
    o2_ref[...] = jax.lax.dot(a, w2_ref[...], precision=jax.lax.Precision.HIGHEST)


def _mm2(a, w1, w2):
    E, K = a.shape
    return pl.pallas_call(
        _mm2_body,
        grid=(E // BE,),
        in_specs=[
            pl.BlockSpec((BE, K), lambda i: (i, 0)),
            pl.BlockSpec((K, w1.shape[1]), lambda i: (0, 0)),
            pl.BlockSpec((K, w2.shape[1]), lambda i: (0, 0)),
        ],
        out_specs=[
            pl.BlockSpec((BE, w1.shape[1]), lambda i: (i, 0)),
            pl.BlockSpec((BE, w2.shape[1]), lambda i: (i, 0)),
        ],
        out_shape=[
            jax.ShapeDtypeStruct((E, w1.shape[1]), jnp.float32),
            jax.ShapeDtypeStruct((E, w2.shape[1]), jnp.float32),
        ],
    )(a, w1, w2)


def _mm1_body(a_ref, w_ref, o_ref):
    o_ref[...] = jax.lax.dot(a_ref[...], w_ref[...],
                             precision=jax.lax.Precision.HIGHEST)


def _mm1(a, w):
    M, K = a.shape
    BN = 2000
    return pl.pallas_call(
        _mm1_body,
        grid=(M // BN,),
        in_specs=[pl.BlockSpec((BN, K), lambda i: (i, 0)),
                  pl.BlockSpec((K, w.shape[1]), lambda i: (0, 0))],
        out_specs=pl.BlockSpec((BN, w.shape[1]), lambda i: (i, 0)),
        out_shape=jax.ShapeDtypeStruct((M, w.shape[1]), jnp.float32),
    )(a, w)


def _prep_body(mp_ref, mr_ref, xa_ref, mp8_ref, mr8_ref, ed_ref, er_ref, ea_ref):
    mpv = jnp.max(mp_ref[...], axis=0, keepdims=True)
    mrv = jnp.max(mr_ref[...], axis=0, keepdims=True)
    mp8_ref[...] = jnp.broadcast_to(mpv, (8, ND))
    mr8_ref[...] = jnp.broadcast_to(mrv, (8, ND))
    xa = xa_ref[...]
    ed_ref[...] = jnp.exp(xa - mpv)
    er_ref[...] = jnp.exp(xa - mrv)
    ea_ref[...] = jnp.exp(xa)


def _prep(mp_p, mr_p, xa):
    N = xa.shape[0]
    return pl.pallas_call(
        _prep_body,
        grid=(1,),
        in_specs=[
            pl.BlockSpec((32, ND), lambda i: (0, 0)),
            pl.BlockSpec((32, ND), lambda i: (0, 0)),
            pl.BlockSpec((N, ND), lambda i: (0, 0)),
        ],
        out_specs=[
            pl.BlockSpec((8, ND), lambda i: (0, 0)),
            pl.BlockSpec((8, ND), lambda i: (0, 0)),
            pl.BlockSpec((N, ND), lambda i: (0, 0)),
            pl.BlockSpec((N, ND), lambda i: (0, 0)),
            pl.BlockSpec((N, ND), lambda i: (0, 0)),
        ],
        out_shape=[
            jax.ShapeDtypeStruct((8, ND), jnp.float32),
            jax.ShapeDtypeStruct((8, ND), jnp.float32),
            jax.ShapeDtypeStruct((N, ND), jnp.float32),
            jax.ShapeDtypeStruct((N, ND), jnp.float32),
            jax.ShapeDtypeStruct((N, ND), jnp.float32),
        ],
    )(mp_p.reshape(32, ND), mr_p.reshape(32, ND), xa)


def _edge_out_body(m_ref, efb_ref, wm_ref, we_ref, ef_ref, conf_ref):
    hi = jax.lax.Precision.HIGHEST
    raw = jax.lax.dot(m_ref[...], wm_ref[...], precision=hi)
    ef2 = _leaky(_leaky(raw + efb_ref[...], ALPHA), 0.1)
    ef_ref[...] = ef2
    conf_ref[...] = jax.lax.dot(ef2, we_ref[...], precision=hi)


def _edge_out(m, efb, wm, we_pad):
    E = efb.shape[0]
    return pl.pallas_call(
        _edge_out_body,
        grid=(E // BE,),
        in_specs=[
            pl.BlockSpec((BE, ND), lambda i: (i, 0)),
            pl.BlockSpec((BE, ED), lambda i: (i, 0)),
            pl.BlockSpec((ND, ED), lambda i: (0, 0)),
            pl.BlockSpec((ED, we_pad.shape[1]), lambda i: (0, 0)),
        ],
        out_specs=[
            pl.BlockSpec((BE, ED), lambda i: (i, 0)),
            pl.BlockSpec((BE, we_pad.shape[1]), lambda i: (i, 0)),
        ],
        out_shape=[
            jax.ShapeDtypeStruct((E, ED), jnp.float32),
            jax.ShapeDtypeStruct((E, we_pad.shape[1]), jnp.float32),
        ],
    )(m, efb, wm, we_pad)


def _node_out_body(s_ref, ea_ref, x_ref, wa_ref, wb_ref, wn_ref,
                   x_new_ref, conf_ref):
    hi = jax.lax.Precision.HIGHEST
    ea = ea_ref[...]
    z = ea * s_ref[0]
    z2 = ea * s_ref[1] / (z + 1e-05)
    ft = (jax.lax.dot(z2, wa_ref[...], precision=hi)
          + jax.lax.dot(x_ref[...], wb_ref[...], precision=hi))
    xn = _leaky(_leaky(ft, ALPHA), 0.1)
    x_new_ref[...] = xn
    conf_ref[...] = jax.lax.dot(xn, wn_ref[...], precision=hi)


def _node_out(S, ea, x, wa, wb, wn_pad):
    N = x.shape[0]
    BN = 2000
    return pl.pallas_call(
        _node_out_body,
        grid=(N // BN,),
        in_specs=[
            pl.BlockSpec((2, BN, ND), lambda i: (0, i, 0)),
            pl.BlockSpec((BN, ND), lambda i: (i, 0)),
            pl.BlockSpec((BN, ND), lambda i: (i, 0)),
            pl.BlockSpec((ED, ND), lambda i: (0, 0)),
            pl.BlockSpec((ND, ND), lambda i: (0, 0)),
            pl.BlockSpec((ND, wn_pad.shape[1]), lambda i: (0, 0)),
        ],
        out_specs=[
            pl.BlockSpec((BN, ND), lambda i: (i, 0)),
            pl.BlockSpec((BN, wn_pad.shape[1]), lambda i: (i, 0)),
        ],
        out_shape=[
            jax.ShapeDtypeStruct((N, ND), jnp.float32),
            jax.ShapeDtypeStruct((N, wn_pad.shape[1]), jnp.float32),
        ],
    )(S, ea, x, wa, wb, wn_pad)


def _sc_max_body(efa, xa, dsti, srci, mp_out, mr_out,
                 efa_b, gd_b, gs_b, di_b, si_b, mx_b, sem):
    c = lax.axis_index("c")
    s = lax.axis_index("s")
    w = c * 16 + s
    e0 = w * (N_EDGES // 32)
    nj = ND // 16
    ninf = jnp.full((16,), -jnp.inf, jnp.float32)

    def chunk(k, acc):
        base = pl.multiple_of(e0 + k * CH, 8)
        pltpu.sync_copy(efa.at[pl.ds(base, CH)], efa_b)
        pltpu.sync_copy(dsti.at[pl.ds(base, CH)], di_b)
        pltpu.sync_copy(srci.at[pl.ds(base, CH)], si_b)
        pltpu.async_copy(xa.at[di_b], gd_b, sem)
        pltpu.async_copy(xa.at[si_b], gs_b, sem).wait()
        pltpu.make_async_copy(xa.at[di_b], gd_b, sem).wait()

        def row(r, a):
            mp, mr = a
            for j in range(nj):
                jsl = pl.ds(16 * j, 16)
                e = efa_b[r, jsl]
                mp = (mp[:j]
                      + (jnp.maximum(mp[j], e + gd_b[r, jsl]),) + mp[j + 1:])
                mr = (mr[:j]
                      + (jnp.maximum(mr[j], e + gs_b[r, jsl]),) + mr[j + 1:])
            return (mp, mr)

        return lax.fori_loop(0, CH, row, acc)

    nk = N_EDGES // 32 // CH
    mp, mr = lax.fori_loop(0, nk, chunk, ((ninf,) * nj, (ninf,) * nj))
    for j in range(nj):
        mx_b[0, pl.ds(16 * j, 16)] = mp[j]
        mx_b[1, pl.ds(16 * j, 16)] = mr[j]
    off = pl.multiple_of(w * ND, 8)
    pltpu.sync_copy(mx_b.at[0], mp_out.at[pl.ds(off, ND)])
    pltpu.sync_copy(mx_b.at[1], mr_out.at[pl.ds(off, ND)])


def _sc_max(efa, xa, dsti, srci):
    f = pl.kernel(
        _sc_max_body,
        out_type=[jax.ShapeDtypeStruct((32 * ND,), jnp.float32),
                  jax.ShapeDtypeStruct((32 * ND,), jnp.float32)],
        mesh=_MESH,
        scratch_types=[
            pltpu.VMEM((CH, ND), jnp.float32),
            pltpu.VMEM((CH, ND), jnp.float32),
            pltpu.VMEM((CH, ND), jnp.float32),
            pltpu.VMEM((CH,), jnp.int32),
            pltpu.VMEM((CH,), jnp.int32),
            pltpu.VMEM((2, ND), jnp.float32),
            pltpu.SemaphoreType.DMA,
        ],
    )
    return f(efa, xa, dsti, srci)


def _sc_zs_body(efa, ef, mp8, dsti, zer,
                s_out,
                acc_sh, efa_b, ef_b, w_b, di_b, mpb, sem):
    c = lax.axis_index("c")
    s = lax.axis_index("s")
    row0 = s * 640
    nst = jnp.where(s < 15, 640 // SR, 400 // SR)

    def zst(i, _):
        r = pl.multiple_of(row0 + i * SR, 8)
        pltpu.sync_copy(zer, acc_sh.at[pl.ds(r, SR)])
        return 0

    lax.fori_loop(0, nst, zst, 0)
    pltpu.sync_copy(mp8, mpb)
    plsc.subcore_barrier()

    nj = ND // 16
    mp = tuple(mpb[0, pl.ds(16 * j, 16)] for j in range(nj))

    e0 = s * (N_EDGES // 16)
    nk = N_EDGES // 16 // CH

    def chunk(k, carry):
        base = pl.multiple_of(e0 + k * CH, 8)
        pltpu.sync_copy(efa.at[pl.ds(base, CH)], efa_b)
        pltpu.sync_copy(dsti.at[pl.ds(base, CH)], di_b)

        @pl.when(c == 1)
        def _():
            pltpu.sync_copy(ef.at[pl.ds(base, CH)], ef_b)

        def row(r, _):
            for j in range(nj):
                jsl = pl.ds(16 * j, 16)
                w_b[r, jsl] = jnp.exp(efa_b[r, jsl] - mp[j])
            return 0

        lax.fori_loop(0, CH, row, 0)

        @pl.when(c == 1)
        def _():
            def row2(r, _):
                for j in range(nj):
                    jsl = pl.ds(16 * j, 16)
                    w_b[r, jsl] = w_b[r, jsl] * ef_b[r, jsl]
                return 0

            lax.fori_loop(0, CH, row2, 0)

        pltpu.sync_copy(w_b, acc_sh.at[di_b], add=True)
        return carry

    lax.fori_loop(0, nk, chunk, 0)
    plsc.subcore_barrier()

    def dmp(i, _):
        r = pl.multiple_of(row0 + i * SR, 8)
        pltpu.sync_copy(acc_sh.at[pl.ds(r, SR)], s_out.at[c, pl.ds(r, SR)])
        return 0

    lax.fori_loop(0, nst, dmp, 0)


def _sc_zs(efa, ef, mp8, dsti, zer):
    f = pl.kernel(
        _sc_zs_body,
        out_type=jax.ShapeDtypeStruct((2, N_NODES, ND), jnp.float32),
        mesh=_MESH,
        scratch_types=[
            pltpu.VMEM_SHARED((N_NODES, ND), jnp.float32),
            pltpu.VMEM((CH, ND), jnp.float32),
            pltpu.VMEM((CH, ND), jnp.float32),
            pltpu.VMEM((CH, ND), jnp.float32),
            pltpu.VMEM((CH,), jnp.int32),
            pltpu.VMEM((8, ND), jnp.float32),
            pltpu.SemaphoreType.DMA,
        ],
    )
    return f(efa, ef, mp8, dsti, zer)


def _sc_m_body(ed, er, x, dsti, srci,
               m_out,
               gpd_b, grs_b, xd_b, xs_b, m_b, di_b, si_b, sem):
    c = lax.axis_index("c")
    s = lax.axis_index("s")
    w = c * 16 + s
    e0 = w * (N_EDGES // 32)
    nj = ND // 16
    nk = N_EDGES // 32 // CH

    def chunk(k, carry):
        base = pl.multiple_of(e0 + k * CH, 8)
        pltpu.sync_copy(dsti.at[pl.ds(base, CH)], di_b)
        pltpu.sync_copy(srci.at[pl.ds(base, CH)], si_b)
        pltpu.async_copy(ed.at[di_b], gpd_b, sem)
        pltpu.async_copy(er.at[si_b], grs_b, sem)
        pltpu.async_copy(x.at[di_b], xd_b, sem)
        pltpu.async_copy(x.at[si_b], xs_b, sem).wait()
        pltpu.make_async_copy(x.at[di_b], xd_b, sem).wait()
        pltpu.make_async_copy(er.at[si_b], grs_b, sem).wait()
        pltpu.make_async_copy(ed.at[di_b], gpd_b, sem).wait()

        def row(r, _):
            for j in range(nj):
                jsl = pl.ds(16 * j, 16)
                ap = gpd_b[r, jsl]
                ar = grs_b[r, jsl]
                m_b[r, jsl] = ((ar * xs_b[r, jsl] + ap * xd_b[r, jsl])
                               / (ap + ar))
            return 0

        lax.fori_loop(0, CH, row, 0)
        pltpu.sync_copy(m_b, m_out.at[pl.ds(base, CH)])
        return carry

    lax.fori_loop(0, nk, chunk, 0)


def _sc_m(ed, er, x, dsti, srci):
    f = pl.kernel(
        _sc_m_body,
        out_type=jax.ShapeDtypeStruct((N_EDGES, ND), jnp.float32),
        mesh=_MESH,
        scratch_types=[
            pltpu.VMEM((CH, ND), jnp.float32),
            pltpu.VMEM((CH, ND), jnp.float32),
            pltpu.VMEM((CH, ND), jnp.float32),
            pltpu.VMEM((CH, ND), jnp.float32),
            pltpu.VMEM((CH, ND), jnp.float32),
            pltpu.VMEM((CH,), jnp.int32),
            pltpu.VMEM((CH,), jnp.int32),
            pltpu.SemaphoreType.DMA,
        ],
    )
    return f(ed, er, x, dsti, srci)


def _layer(x, ef, dsti, srci, fa, fnup, feup, wn_pad, we_pad, zer):
    faE, faX = fa[:ED], fa[ED:]
    fnupA, fnupB = fnup[:ED], fnup[ED:]
    feupM = feup[:ND]
    feupE = feup[ND:]

    efa, efb = _mm2(ef, faE, feupE)
    xa = _mm1(x, faX)

    mp_p, mr_p = _sc_max(efa, xa, dsti, srci)
    mp8, mr8, ed, er, ea = _prep(mp_p, mr_p, xa)
    S = _sc_zs(efa, ef, mp8, dsti, zer)
    m = _sc_m(ed, er, x, dsti, srci)

    x_new, node_conf = _node_out(S, ea, x, fnupA, fnupB, wn_pad)
    ef_new, edge_conf = _edge_out(m, efb, feupM, we_pad)
    return x_new, ef_new, node_conf, edge_conf


def kernel(x, edge_feats, fa0, fnup0, feup0, fa1, fnup1, feup1, Wn, We, edge_index):
    srci = edge_index[0]
    dsti = edge_index[1]
    wn_pad = jnp.pad(Wn, ((0, 0), (0, 64 - Wn.shape[1])))
    we_pad = jnp.pad(We, ((0, 0), (0, 8 - We.shape[1])))
    zer = jnp.zeros((SR, ND), jnp.float32)

    nf, ef = x, edge_feats
    nf, ef, _, _ = _layer(nf, ef, dsti, srci, fa0, fnup0, feup0,
                          wn_pad, we_pad, zer)
    nf, ef, node_conf, edge_conf = _layer(nf, ef, dsti, srci, fa1, fnup1,
                                          feup1, wn_pad, we_pad, zer)
    return (nf, ef, node_conf[:, :Wn.shape[1]], edge_conf[:, :We.shape[1]])


# speedup vs baseline: 2.0826x; 1.1833x over previous
import functools

import jax
import jax.numpy as jnp
from jax import lax
from jax.experimental import pallas as pl
from jax.experimental.pallas import tpu as pltpu
from jax.experimental.pallas import tpu_sc as plsc

N_NODES = 10000
N_EDGES = 320000
ND = 128
ED = 128
ALPHA = 0.1
BE = 2000
CH = 80
SR = 80

_MESH = plsc.VectorSubcoreMesh(core_axis_name="c", subcore_axis_name="s")


def _leaky(v, a):
    return jnp.where(v > 0, v, a * v)


def _mm2_body(a_ref, w1_ref, w2_ref, o1_ref, o2_ref):
    a = a_ref[...]
    o1_ref[...] = jax.lax.dot(a, w1_ref[...], precision=None)
    o2_ref[...] = jax.lax.dot(a, w2_ref[...], precision=None)


def _mm2(a, w1, w2):
    E, K = a.shape
    return pl.pallas_call(
        _mm2_body,
        grid=(E // BE,),
        in_specs=[
            pl.BlockSpec((BE, K), lambda i: (i, 0)),
            pl.BlockSpec((K, w1.shape[1]), lambda i: (0, 0)),
            pl.BlockSpec((K, w2.shape[1]), lambda i: (0, 0)),
        ],
        out_specs=[
            pl.BlockSpec((BE, w1.shape[1]), lambda i: (i, 0)),
            pl.BlockSpec((BE, w2.shape[1]), lambda i: (i, 0)),
        ],
        out_shape=[
            jax.ShapeDtypeStruct((E, w1.shape[1]), jnp.float32),
            jax.ShapeDtypeStruct((E, w2.shape[1]), jnp.float32),
        ],
    )(a, w1, w2)


def _mm1_body(a_ref, w_ref, o_ref):
    o_ref[...] = jax.lax.dot(a_ref[...], w_ref[...],
                             precision=None)


def _mm1(a, w):
    M, K = a.shape
    BN = 2000
    return pl.pallas_call(
        _mm1_body,
        grid=(M // BN,),
        in_specs=[pl.BlockSpec((BN, K), lambda i: (i, 0)),
                  pl.BlockSpec((K, w.shape[1]), lambda i: (0, 0))],
        out_specs=pl.BlockSpec((BN, w.shape[1]), lambda i: (i, 0)),
        out_shape=jax.ShapeDtypeStruct((M, w.shape[1]), jnp.float32),
    )(a, w)


def _prep_body(mp_ref, mr_ref, xa_ref, mp8_ref, mr8_ref, ed_ref, er_ref, ea_ref):
    mpv = jnp.max(mp_ref[...], axis=0, keepdims=True)
    mrv = jnp.max(mr_ref[...], axis=0, keepdims=True)
    mp8_ref[...] = jnp.broadcast_to(mpv, (8, ND))
    mr8_ref[...] = jnp.broadcast_to(mrv, (8, ND))
    xa = xa_ref[...]
    ed_ref[...] = jnp.exp(xa - mpv)
    er_ref[...] = jnp.exp(xa - mrv)
    ea_ref[...] = jnp.exp(xa)


def _prep(mp_p, mr_p, xa):
    N = xa.shape[0]
    return pl.pallas_call(
        _prep_body,
        grid=(1,),
        in_specs=[
            pl.BlockSpec((32, ND), lambda i: (0, 0)),
            pl.BlockSpec((32, ND), lambda i: (0, 0)),
            pl.BlockSpec((N, ND), lambda i: (0, 0)),
        ],
        out_specs=[
            pl.BlockSpec((8, ND), lambda i: (0, 0)),
            pl.BlockSpec((8, ND), lambda i: (0, 0)),
            pl.BlockSpec((N, ND), lambda i: (0, 0)),
            pl.BlockSpec((N, ND), lambda i: (0, 0)),
            pl.BlockSpec((N, ND), lambda i: (0, 0)),
        ],
        out_shape=[
            jax.ShapeDtypeStruct((8, ND), jnp.float32),
            jax.ShapeDtypeStruct((8, ND), jnp.float32),
            jax.ShapeDtypeStruct((N, ND), jnp.float32),
            jax.ShapeDtypeStruct((N, ND), jnp.float32),
            jax.ShapeDtypeStruct((N, ND), jnp.float32),
        ],
    )(mp_p.reshape(32, ND), mr_p.reshape(32, ND), xa)


def _edge_out_body(m_ref, efb_ref, wm_ref, we_ref, ef_ref, conf_ref):
    hi = None
    raw = jax.lax.dot(m_ref[...], wm_ref[...], precision=hi)
    ef2 = _leaky(_leaky(raw + efb_ref[...], ALPHA), 0.1)
    ef_ref[...] = ef2
    conf_ref[...] = jax.lax.dot(ef2, we_ref[...], precision=hi)


def _edge_out(m, efb, wm, we_pad):
    E = efb.shape[0]
    return pl.pallas_call(
        _edge_out_body,
        grid=(E // BE,),
        in_specs=[
            pl.BlockSpec((BE, ND), lambda i: (i, 0)),
            pl.BlockSpec((BE, ED), lambda i: (i, 0)),
            pl.BlockSpec((ND, ED), lambda i: (0, 0)),
            pl.BlockSpec((ED, we_pad.shape[1]), lambda i: (0, 0)),
        ],
        out_specs=[
            pl.BlockSpec((BE, ED), lambda i: (i, 0)),
            pl.BlockSpec((BE, we_pad.shape[1]), lambda i: (i, 0)),
        ],
        out_shape=[
            jax.ShapeDtypeStruct((E, ED), jnp.float32),
            jax.ShapeDtypeStruct((E, we_pad.shape[1]), jnp.float32),
        ],
    )(m, efb, wm, we_pad)


def _node_out_body(s_ref, ea_ref, x_ref, wa_ref, wb_ref, wn_ref,
                   x_new_ref, conf_ref):
    hi = None
    ea = ea_ref[...]
    z = ea * s_ref[0]
    z2 = ea * s_ref[1] / (z + 1e-05)
    ft = (jax.lax.dot(z2, wa_ref[...], precision=hi)
          + jax.lax.dot(x_ref[...], wb_ref[...], precision=hi))
    xn = _leaky(_leaky(ft, ALPHA), 0.1)
    x_new_ref[...] = xn
    conf_ref[...] = jax.lax.dot(xn, wn_ref[...], precision=hi)


def _node_out(S, ea, x, wa, wb, wn_pad):
    N = x.shape[0]
    BN = 2000
    return pl.pallas_call(
        _node_out_body,
        grid=(N // BN,),
        in_specs=[
            pl.BlockSpec((2, BN, ND), lambda i: (0, i, 0)),
            pl.BlockSpec((BN, ND), lambda i: (i, 0)),
            pl.BlockSpec((BN, ND), lambda i: (i, 0)),
            pl.BlockSpec((ED, ND), lambda i: (0, 0)),
            pl.BlockSpec((ND, ND), lambda i: (0, 0)),
            pl.BlockSpec((ND, wn_pad.shape[1]), lambda i: (0, 0)),
        ],
        out_specs=[
            pl.BlockSpec((BN, ND), lambda i: (i, 0)),
            pl.BlockSpec((BN, wn_pad.shape[1]), lambda i: (i, 0)),
        ],
        out_shape=[
            jax.ShapeDtypeStruct((N, ND), jnp.float32),
            jax.ShapeDtypeStruct((N, wn_pad.shape[1]), jnp.float32),
        ],
    )(S, ea, x, wa, wb, wn_pad)


def _sc_max_body(efa, xa, dsti, srci, mp_out, mr_out,
                 efa_b, gd_b, gs_b, di_b, si_b, mx_b, sem):
    c = lax.axis_index("c")
    s = lax.axis_index("s")
    w = c * 16 + s
    e0 = w * (N_EDGES // 32)
    nj = ND // 16
    ninf = jnp.full((16,), -jnp.inf, jnp.float32)

    def chunk(k, acc):
        base = pl.multiple_of(e0 + k * CH, 8)
        pltpu.sync_copy(efa.at[pl.ds(base, CH)], efa_b)
        pltpu.sync_copy(dsti.at[pl.ds(base, CH)], di_b)
        pltpu.sync_copy(srci.at[pl.ds(base, CH)], si_b)
        pltpu.async_copy(xa.at[di_b], gd_b, sem)
        pltpu.async_copy(xa.at[si_b], gs_b, sem).wait()
        pltpu.make_async_copy(xa.at[di_b], gd_b, sem).wait()

        def row(r, a):
            mp, mr = a
            for j in range(nj):
                jsl = pl.ds(16 * j, 16)
                e = efa_b[r, jsl]
                mp = (mp[:j]
                      + (jnp.maximum(mp[j], e + gd_b[r, jsl]),) + mp[j + 1:])
                mr = (mr[:j]
                      + (jnp.maximum(mr[j], e + gs_b[r, jsl]),) + mr[j + 1:])
            return (mp, mr)

        return lax.fori_loop(0, CH, row, acc)

    nk = N_EDGES // 32 // CH
    mp, mr = lax.fori_loop(0, nk, chunk, ((ninf,) * nj, (ninf,) * nj))
    for j in range(nj):
        mx_b[0, pl.ds(16 * j, 16)] = mp[j]
        mx_b[1, pl.ds(16 * j, 16)] = mr[j]
    off = pl.multiple_of(w * ND, 8)
    pltpu.sync_copy(mx_b.at[0], mp_out.at[pl.ds(off, ND)])
    pltpu.sync_copy(mx_b.at[1], mr_out.at[pl.ds(off, ND)])


def _sc_max(efa, xa, dsti, srci):
    f = pl.kernel(
        _sc_max_body,
        out_type=[jax.ShapeDtypeStruct((32 * ND,), jnp.float32),
                  jax.ShapeDtypeStruct((32 * ND,), jnp.float32)],
        mesh=_MESH,
        scratch_types=[
            pltpu.VMEM((CH, ND), jnp.float32),
            pltpu.VMEM((CH, ND), jnp.float32),
            pltpu.VMEM((CH, ND), jnp.float32),
            pltpu.VMEM((CH,), jnp.int32),
            pltpu.VMEM((CH,), jnp.int32),
            pltpu.VMEM((2, ND), jnp.float32),
            pltpu.SemaphoreType.DMA,
        ],
    )
    return f(efa, xa, dsti, srci)


def _sc_zs_body(efa, ef, mp8, dsti, zer,
                s_out,
                acc_sh, efa_b, ef_b, w_b, di_b, mpb, sem):
    c = lax.axis_index("c")
    s = lax.axis_index("s")
    row0 = s * 640
    nst = jnp.where(s < 15, 640 // SR, 400 // SR)

    def zst(i, _):
        r = pl.multiple_of(row0 + i * SR, 8)
        pltpu.sync_copy(zer, acc_sh.at[pl.ds(r, SR)])
        return 0

    lax.fori_loop(0, nst, zst, 0)
    pltpu.sync_copy(mp8, mpb)
    plsc.subcore_barrier()

    nj = ND // 16
    mp = tuple(mpb[0, pl.ds(16 * j, 16)] for j in range(nj))

    e0 = s * (N_EDGES // 16)
    nk = N_EDGES // 16 // CH

    def chunk(k, carry):
        base = pl.multiple_of(e0 + k * CH, 8)
        pltpu.sync_copy(efa.at[pl.ds(base, CH)], efa_b)
        pltpu.sync_copy(dsti.at[pl.ds(base, CH)], di_b)

        @pl.when(c == 1)
        def _():
            pltpu.sync_copy(ef.at[pl.ds(base, CH)], ef_b)

        @pl.when(c == 0)
        def _():
            def row(r, _):
                for j in range(nj):
                    jsl = pl.ds(16 * j, 16)
                    w_b[r, jsl] = jnp.exp(efa_b[r, jsl] - mp[j])
                return 0

            lax.fori_loop(0, CH, row, 0)

        @pl.when(c == 1)
        def _():
            def row(r, _):
                for j in range(nj):
                    jsl = pl.ds(16 * j, 16)
                    w_b[r, jsl] = (jnp.exp(efa_b[r, jsl] - mp[j])
                                   * ef_b[r, jsl])
                return 0

            lax.fori_loop(0, CH, row, 0)

        pltpu.sync_copy(w_b, acc_sh.at[di_b], add=True)
        return carry

    lax.fori_loop(0, nk, chunk, 0)
    plsc.subcore_barrier()

    def dmp(i, _):
        r = pl.multiple_of(row0 + i * SR, 8)
        pltpu.sync_copy(acc_sh.at[pl.ds(r, SR)], s_out.at[c, pl.ds(r, SR)])
        return 0

    lax.fori_loop(0, nst, dmp, 0)


def _sc_zs(efa, ef, mp8, dsti, zer):
    f = pl.kernel(
        _sc_zs_body,
        out_type=jax.ShapeDtypeStruct((2, N_NODES, ND), jnp.float32),
        mesh=_MESH,
        scratch_types=[
            pltpu.VMEM_SHARED((N_NODES, ND), jnp.float32),
            pltpu.VMEM((CH, ND), jnp.float32),
            pltpu.VMEM((CH, ND), jnp.float32),
            pltpu.VMEM((CH, ND), jnp.float32),
            pltpu.VMEM((CH,), jnp.int32),
            pltpu.VMEM((8, ND), jnp.float32),
            pltpu.SemaphoreType.DMA,
        ],
    )
    return f(efa, ef, mp8, dsti, zer)


def _sc_m_body(ed, er, x, dsti, srci,
               m_out,
               gpd_b, grs_b, xd_b, xs_b, m_b, di_b, si_b, sem):
    c = lax.axis_index("c")
    s = lax.axis_index("s")
    w = c * 16 + s
    e0 = w * (N_EDGES // 32)
    nj = ND // 16
    nk = N_EDGES // 32 // CH

    def chunk(k, carry):
        base = pl.multiple_of(e0 + k * CH, 8)
        pltpu.sync_copy(dsti.at[pl.ds(base, CH)], di_b)
        pltpu.sync_copy(srci.at[pl.ds(base, CH)], si_b)
        pltpu.async_copy(ed.at[di_b], gpd_b, sem)
        pltpu.async_copy(er.at[si_b], grs_b, sem)
        pltpu.async_copy(x.at[di_b], xd_b, sem)
        pltpu.async_copy(x.at[si_b], xs_b, sem).wait()
        pltpu.make_async_copy(x.at[di_b], xd_b, sem).wait()
        pltpu.make_async_copy(er.at[si_b], grs_b, sem).wait()
        pltpu.make_async_copy(ed.at[di_b], gpd_b, sem).wait()

        def row(r, _):
            for j in range(nj):
                jsl = pl.ds(16 * j, 16)
                ap = gpd_b[r, jsl]
                ar = grs_b[r, jsl]
                m_b[r, jsl] = ((ar * xs_b[r, jsl] + ap * xd_b[r, jsl])
                               / (ap + ar))
            return 0

        lax.fori_loop(0, CH, row, 0)
        pltpu.sync_copy(m_b, m_out.at[pl.ds(base, CH)])
        return carry

    lax.fori_loop(0, nk, chunk, 0)


def _sc_m(ed, er, x, dsti, srci):
    f = pl.kernel(
        _sc_m_body,
        out_type=jax.ShapeDtypeStruct((N_EDGES, ND), jnp.float32),
        mesh=_MESH,
        scratch_types=[
            pltpu.VMEM((CH, ND), jnp.float32),
            pltpu.VMEM((CH, ND), jnp.float32),
            pltpu.VMEM((CH, ND), jnp.float32),
            pltpu.VMEM((CH, ND), jnp.float32),
            pltpu.VMEM((CH, ND), jnp.float32),
            pltpu.VMEM((CH,), jnp.int32),
            pltpu.VMEM((CH,), jnp.int32),
            pltpu.SemaphoreType.DMA,
        ],
    )
    return f(ed, er, x, dsti, srci)


def _layer(x, ef, dsti, srci, fa, fnup, feup, wn_pad, we_pad, zer):
    faE, faX = fa[:ED], fa[ED:]
    fnupA, fnupB = fnup[:ED], fnup[ED:]
    feupM = feup[:ND]
    feupE = feup[ND:]

    efa, efb = _mm2(ef, faE, feupE)
    xa = _mm1(x, faX)

    mp_p, mr_p = _sc_max(efa, xa, dsti, srci)
    mp8, mr8, ed, er, ea = _prep(mp_p, mr_p, xa)
    S = _sc_zs(efa, ef, mp8, dsti, zer)
    m = _sc_m(ed, er, x, dsti, srci)

    x_new, node_conf = _node_out(S, ea, x, fnupA, fnupB, wn_pad)
    ef_new, edge_conf = _edge_out(m, efb, feupM, we_pad)
    return x_new, ef_new, node_conf, edge_conf


def kernel(x, edge_feats, fa0, fnup0, feup0, fa1, fnup1, feup1, Wn, We, edge_index):
    srci = edge_index[0]
    dsti = edge_index[1]
    wn_pad = jnp.pad(Wn, ((0, 0), (0, 64 - Wn.shape[1])))
    we_pad = jnp.pad(We, ((0, 0), (0, 8 - We.shape[1])))
    zer = jnp.zeros((SR, ND), jnp.float32)

    nf, ef = x, edge_feats
    nf, ef, _, _ = _layer(nf, ef, dsti, srci, fa0, fnup0, feup0,
                          wn_pad, we_pad, zer)
    nf, ef, node_conf, edge_conf = _layer(nf, ef, dsti, srci, fa1, fnup1,
                                          feup1, wn_pad, we_pad, zer)
    return (nf, ef, node_conf[:, :Wn.shape[1]], edge_conf[:, :We.shape[1]])


# speedup vs baseline: 2.4606x; 1.1815x over previous
import functools

import jax
import jax.numpy as jnp
from jax import lax
from jax.experimental import pallas as pl
from jax.experimental.pallas import tpu as pltpu
from jax.experimental.pallas import tpu_sc as plsc

N_NODES = 10000
N_EDGES = 320000
ND = 128
ED = 128
ALPHA = 0.1
BE = 2000
CH = 80
SR = 80

_MESH = plsc.VectorSubcoreMesh(core_axis_name="c", subcore_axis_name="s")


def _leaky(v, a):
    return jnp.where(v > 0, v, a * v)


def _mm2_body(a_ref, w1_ref, w2_ref, o1_ref, o2_ref):
    a = a_ref[...]
    o1_ref[...] = jax.lax.dot(a, w1_ref[...], precision=None)
    o2_ref[...] = jax.lax.dot(a, w2_ref[...], precision=None)


def _mm2(a, w1, w2):
    E, K = a.shape
    return pl.pallas_call(
        _mm2_body,
        grid=(E // BE,),
        in_specs=[
            pl.BlockSpec((BE, K), lambda i: (i, 0)),
            pl.BlockSpec((K, w1.shape[1]), lambda i: (0, 0)),
            pl.BlockSpec((K, w2.shape[1]), lambda i: (0, 0)),
        ],
        out_specs=[
            pl.BlockSpec((BE, w1.shape[1]), lambda i: (i, 0)),
            pl.BlockSpec((BE, w2.shape[1]), lambda i: (i, 0)),
        ],
        out_shape=[
            jax.ShapeDtypeStruct((E, w1.shape[1]), jnp.float32),
            jax.ShapeDtypeStruct((E, w2.shape[1]), jnp.float32),
        ],
    )(a, w1, w2)


def _mm1_body(a_ref, w_ref, o_ref):
    o_ref[...] = jax.lax.dot(a_ref[...], w_ref[...],
                             precision=None)


def _mm1(a, w):
    M, K = a.shape
    BN = 2000
    return pl.pallas_call(
        _mm1_body,
        grid=(M // BN,),
        in_specs=[pl.BlockSpec((BN, K), lambda i: (i, 0)),
                  pl.BlockSpec((K, w.shape[1]), lambda i: (0, 0))],
        out_specs=pl.BlockSpec((BN, w.shape[1]), lambda i: (i, 0)),
        out_shape=jax.ShapeDtypeStruct((M, w.shape[1]), jnp.float32),
    )(a, w)


def _prep_body(mp_ref, mr_ref, xa_ref, mp8_ref, mr8_ref, ed_ref, er_ref, ea_ref):
    mpv = jnp.max(mp_ref[...], axis=0, keepdims=True)
    mrv = jnp.max(mr_ref[...], axis=0, keepdims=True)
    mp8_ref[...] = jnp.broadcast_to(mpv, (8, ND))
    mr8_ref[...] = jnp.broadcast_to(mrv, (8, ND))
    xa = xa_ref[...]
    ed_ref[...] = jnp.exp(xa - mpv)
    er_ref[...] = jnp.exp(xa - mrv)
    ea_ref[...] = jnp.exp(xa)


def _prep(mp_p, mr_p, xa):
    N = xa.shape[0]
    return pl.pallas_call(
        _prep_body,
        grid=(1,),
        in_specs=[
            pl.BlockSpec((32, ND), lambda i: (0, 0)),
            pl.BlockSpec((32, ND), lambda i: (0, 0)),
            pl.BlockSpec((N, ND), lambda i: (0, 0)),
        ],
        out_specs=[
            pl.BlockSpec((8, ND), lambda i: (0, 0)),
            pl.BlockSpec((8, ND), lambda i: (0, 0)),
            pl.BlockSpec((N, ND), lambda i: (0, 0)),
            pl.BlockSpec((N, ND), lambda i: (0, 0)),
            pl.BlockSpec((N, ND), lambda i: (0, 0)),
        ],
        out_shape=[
            jax.ShapeDtypeStruct((8, ND), jnp.float32),
            jax.ShapeDtypeStruct((8, ND), jnp.float32),
            jax.ShapeDtypeStruct((N, ND), jnp.float32),
            jax.ShapeDtypeStruct((N, ND), jnp.float32),
            jax.ShapeDtypeStruct((N, ND), jnp.float32),
        ],
    )(mp_p.reshape(32, ND), mr_p.reshape(32, ND), xa)


def _edge_out_body(m_ref, efb_ref, wm_ref, we_ref, ef_ref, conf_ref):
    hi = None
    raw = jax.lax.dot(m_ref[...], wm_ref[...], precision=hi)
    ef2 = _leaky(_leaky(raw + efb_ref[...], ALPHA), 0.1)
    ef_ref[...] = ef2
    conf_ref[...] = jax.lax.dot(ef2, we_ref[...], precision=hi)


def _edge_out(m, efb, wm, we_pad):
    E = efb.shape[0]
    return pl.pallas_call(
        _edge_out_body,
        grid=(E // BE,),
        in_specs=[
            pl.BlockSpec((BE, ND), lambda i: (i, 0)),
            pl.BlockSpec((BE, ED), lambda i: (i, 0)),
            pl.BlockSpec((ND, ED), lambda i: (0, 0)),
            pl.BlockSpec((ED, we_pad.shape[1]), lambda i: (0, 0)),
        ],
        out_specs=[
            pl.BlockSpec((BE, ED), lambda i: (i, 0)),
            pl.BlockSpec((BE, we_pad.shape[1]), lambda i: (i, 0)),
        ],
        out_shape=[
            jax.ShapeDtypeStruct((E, ED), jnp.float32),
            jax.ShapeDtypeStruct((E, we_pad.shape[1]), jnp.float32),
        ],
    )(m, efb, wm, we_pad)


def _node_out_body(s_ref, ea_ref, x_ref, wa_ref, wb_ref, wn_ref,
                   x_new_ref, conf_ref):
    hi = None
    ea = ea_ref[...]
    z = ea * s_ref[0]
    z2 = ea * s_ref[1] / (z + 1e-05)
    ft = (jax.lax.dot(z2, wa_ref[...], precision=hi)
          + jax.lax.dot(x_ref[...], wb_ref[...], precision=hi))
    xn = _leaky(_leaky(ft, ALPHA), 0.1)
    x_new_ref[...] = xn
    conf_ref[...] = jax.lax.dot(xn, wn_ref[...], precision=hi)


def _node_out(S, ea, x, wa, wb, wn_pad):
    N = x.shape[0]
    BN = 2000
    return pl.pallas_call(
        _node_out_body,
        grid=(N // BN,),
        in_specs=[
            pl.BlockSpec((2, BN, ND), lambda i: (0, i, 0)),
            pl.BlockSpec((BN, ND), lambda i: (i, 0)),
            pl.BlockSpec((BN, ND), lambda i: (i, 0)),
            pl.BlockSpec((ED, ND), lambda i: (0, 0)),
            pl.BlockSpec((ND, ND), lambda i: (0, 0)),
            pl.BlockSpec((ND, wn_pad.shape[1]), lambda i: (0, 0)),
        ],
        out_specs=[
            pl.BlockSpec((BN, ND), lambda i: (i, 0)),
            pl.BlockSpec((BN, wn_pad.shape[1]), lambda i: (i, 0)),
        ],
        out_shape=[
            jax.ShapeDtypeStruct((N, ND), jnp.float32),
            jax.ShapeDtypeStruct((N, wn_pad.shape[1]), jnp.float32),
        ],
    )(S, ea, x, wa, wb, wn_pad)


def _sc_max_body(efa, xa, dsti, srci, mp_out, mr_out,
                 efa_b, gd_b, gs_b, di_b, si_b, mx_b, sem):
    c = lax.axis_index("c")
    s = lax.axis_index("s")
    w = c * 16 + s
    e0 = w * (N_EDGES // 32)
    nj = ND // 16
    ninf = jnp.full((16,), -jnp.inf, jnp.float32)

    def chunk(k, acc):
        base = pl.multiple_of(e0 + k * CH, 8)
        pltpu.sync_copy(efa.at[pl.ds(base, CH)], efa_b)
        pltpu.sync_copy(dsti.at[pl.ds(base, CH)], di_b)
        pltpu.sync_copy(srci.at[pl.ds(base, CH)], si_b)
        pltpu.async_copy(xa.at[di_b], gd_b, sem)
        pltpu.async_copy(xa.at[si_b], gs_b, sem).wait()
        pltpu.make_async_copy(xa.at[di_b], gd_b, sem).wait()

        def row(r, a):
            mp, mr = a
            for j in range(nj):
                jsl = pl.ds(16 * j, 16)
                e = efa_b[r, jsl]
                mp = (mp[:j]
                      + (jnp.maximum(mp[j], e + gd_b[r, jsl]),) + mp[j + 1:])
                mr = (mr[:j]
                      + (jnp.maximum(mr[j], e + gs_b[r, jsl]),) + mr[j + 1:])
            return (mp, mr)

        return lax.fori_loop(0, CH, row, acc)

    nk = N_EDGES // 32 // CH
    mp, mr = lax.fori_loop(0, nk, chunk, ((ninf,) * nj, (ninf,) * nj))
    for j in range(nj):
        mx_b[0, pl.ds(16 * j, 16)] = mp[j]
        mx_b[1, pl.ds(16 * j, 16)] = mr[j]
    off = pl.multiple_of(w * ND, 8)
    pltpu.sync_copy(mx_b.at[0], mp_out.at[pl.ds(off, ND)])
    pltpu.sync_copy(mx_b.at[1], mr_out.at[pl.ds(off, ND)])


def _sc_max(efa, xa, dsti, srci):
    f = pl.kernel(
        _sc_max_body,
        out_type=[jax.ShapeDtypeStruct((32 * ND,), jnp.float32),
                  jax.ShapeDtypeStruct((32 * ND,), jnp.float32)],
        mesh=_MESH,
        scratch_types=[
            pltpu.VMEM((CH, ND), jnp.float32),
            pltpu.VMEM((CH, ND), jnp.float32),
            pltpu.VMEM((CH, ND), jnp.float32),
            pltpu.VMEM((CH,), jnp.int32),
            pltpu.VMEM((CH,), jnp.int32),
            pltpu.VMEM((2, ND), jnp.float32),
            pltpu.SemaphoreType.DMA,
        ],
    )
    return f(efa, xa, dsti, srci)


def _sc_zs_body(efa, ef, mp8, dsti, zer,
                s_out,
                acc_sh,
                efa_b0, efa_b1, ef_b0, ef_b1,
                di_b0, di_b1, mpb, sem0, sem1, sems0, sems1):
    c = lax.axis_index("c")
    s = lax.axis_index("s")
    row0 = s * 640
    nst = jnp.where(s < 15, 640 // SR, 400 // SR)

    def zst(i, _):
        r = pl.multiple_of(row0 + i * SR, 8)
        pltpu.sync_copy(zer, acc_sh.at[pl.ds(r, SR)])
        return 0

    lax.fori_loop(0, nst, zst, 0)
    pltpu.sync_copy(mp8.at[0], mpb)
    plsc.subcore_barrier()

    nj = ND // 16
    mp = tuple(mpb[pl.ds(16 * j, 16)] for j in range(nj))

    e0 = s * (N_EDGES // 16)
    nk = N_EDGES // 16 // CH

    bufs = ((efa_b0, ef_b0, efa_b0, di_b0, sem0, sems0),
            (efa_b1, ef_b1, efa_b1, di_b1, sem1, sems1))

    def fire_lin(k, b):
        efa_b, ef_b, _, di_b, sem, _ = bufs[b]
        base = pl.multiple_of(e0 + k * CH, 8)
        pltpu.async_copy(efa.at[pl.ds(base, CH)], efa_b, sem)
        pltpu.async_copy(dsti.at[pl.ds(base, CH)], di_b, sem)

        @pl.when(c == 1)
        def _():
            pltpu.async_copy(ef.at[pl.ds(base, CH)], ef_b, sem)

    def wait_lin(k, b):
        efa_b, ef_b, _, di_b, sem, _ = bufs[b]
        base = pl.multiple_of(e0 + k * CH, 8)
        pltpu.make_async_copy(efa.at[pl.ds(base, CH)], efa_b, sem).wait()
        pltpu.make_async_copy(dsti.at[pl.ds(base, CH)], di_b, sem).wait()

        @pl.when(c == 1)
        def _():
            pltpu.make_async_copy(ef.at[pl.ds(base, CH)], ef_b, sem).wait()

    def compute(b):
        efa_b, ef_b, w_b, _, _, _ = bufs[b]

        @pl.when(c == 0)
        def _():
            def row(r, _):
                for j in range(nj):
                    jsl = pl.ds(16 * j, 16)
                    w_b[r, jsl] = jnp.exp(efa_b[r, jsl] - mp[j])
                return 0

            lax.fori_loop(0, CH, row, 0)

        @pl.when(c == 1)
        def _():
            def row(r, _):
                for j in range(nj):
                    jsl = pl.ds(16 * j, 16)
                    w_b[r, jsl] = (jnp.exp(efa_b[r, jsl] - mp[j])
                                   * ef_b[r, jsl])
                return 0

            lax.fori_loop(0, CH, row, 0)

    def fire_scatter(b):
        _, _, w_b, di_b, _, sems = bufs[b]
        pltpu.async_copy(w_b, acc_sh.at[di_b], sems, add=True)

    def drain_scatter(b):
        _, _, w_b, di_b, _, sems = bufs[b]
        pltpu.make_async_copy(w_b, acc_sh.at[di_b], sems).wait()

    fire_lin(0, 0)

    def pair(gg, carry):
        k0 = gg * 2

        @pl.when(gg > 0)
        def _():
            drain_scatter(1)

        fire_lin(k0 + 1, 1)
        wait_lin(k0, 0)
        compute(0)
        fire_scatter(0)
        wait_lin(k0 + 1, 1)
        compute(1)
        fire_scatter(1)
        drain_scatter(0)

        @pl.when(gg < (nk // 2 - 1))
        def _():
            fire_lin(k0 + 2, 0)

        return carry

    lax.fori_loop(0, nk // 2, pair, 0)
    drain_scatter(1)
    plsc.subcore_barrier()

    def dmp(i, _):
        r = pl.multiple_of(row0 + i * SR, 8)
        pltpu.sync_copy(acc_sh.at[pl.ds(r, SR)], s_out.at[c, pl.ds(r, SR)])
        return 0

    lax.fori_loop(0, nst, dmp, 0)


def _sc_zs(efa, ef, mp8, dsti, zer):
    f = pl.kernel(
        _sc_zs_body,
        out_type=jax.ShapeDtypeStruct((2, N_NODES, ND), jnp.float32),
        mesh=_MESH,
        scratch_types=[
            pltpu.VMEM_SHARED((N_NODES, ND), jnp.float32),
            pltpu.VMEM((CH, ND), jnp.float32),
            pltpu.VMEM((CH, ND), jnp.float32),
            pltpu.VMEM((CH, ND), jnp.float32),
            pltpu.VMEM((CH, ND), jnp.float32),
            pltpu.VMEM((CH,), jnp.int32),
            pltpu.VMEM((CH,), jnp.int32),
            pltpu.VMEM((ND,), jnp.float32),
            pltpu.SemaphoreType.DMA,
            pltpu.SemaphoreType.DMA,
            pltpu.SemaphoreType.DMA,
            pltpu.SemaphoreType.DMA,
        ],
    )
    return f(efa, ef, mp8, dsti, zer)


def _sc_m_body(ed, er, x, dsti, srci,
               m_out,
               gpd_b, grs_b, xd_b, xs_b, m_b, di_b, si_b, sem):
    c = lax.axis_index("c")
    s = lax.axis_index("s")
    w = c * 16 + s
    e0 = w * (N_EDGES // 32)
    nj = ND // 16
    nk = N_EDGES // 32 // CH

    def chunk(k, carry):
        base = pl.multiple_of(e0 + k * CH, 8)
        pltpu.sync_copy(dsti.at[pl.ds(base, CH)], di_b)
        pltpu.sync_copy(srci.at[pl.ds(base, CH)], si_b)
        pltpu.async_copy(ed.at[di_b], gpd_b, sem)
        pltpu.async_copy(er.at[si_b], grs_b, sem)
        pltpu.async_copy(x.at[di_b], xd_b, sem)
        pltpu.async_copy(x.at[si_b], xs_b, sem).wait()
        pltpu.make_async_copy(x.at[di_b], xd_b, sem).wait()
        pltpu.make_async_copy(er.at[si_b], grs_b, sem).wait()
        pltpu.make_async_copy(ed.at[di_b], gpd_b, sem).wait()

        def row(r, _):
            for j in range(nj):
                jsl = pl.ds(16 * j, 16)
                ap = gpd_b[r, jsl]
                ar = grs_b[r, jsl]
                m_b[r, jsl] = ((ar * xs_b[r, jsl] + ap * xd_b[r, jsl])
                               / (ap + ar))
            return 0

        lax.fori_loop(0, CH, row, 0)
        pltpu.sync_copy(m_b, m_out.at[pl.ds(base, CH)])
        return carry

    lax.fori_loop(0, nk, chunk, 0)


def _sc_m(ed, er, x, dsti, srci):
    f = pl.kernel(
        _sc_m_body,
        out_type=jax.ShapeDtypeStruct((N_EDGES, ND), jnp.float32),
        mesh=_MESH,
        scratch_types=[
            pltpu.VMEM((CH, ND), jnp.float32),
            pltpu.VMEM((CH, ND), jnp.float32),
            pltpu.VMEM((CH, ND), jnp.float32),
            pltpu.VMEM((CH, ND), jnp.float32),
            pltpu.VMEM((CH, ND), jnp.float32),
            pltpu.VMEM((CH,), jnp.int32),
            pltpu.VMEM((CH,), jnp.int32),
            pltpu.SemaphoreType.DMA,
        ],
    )
    return f(ed, er, x, dsti, srci)


def _layer(x, ef, dsti, srci, fa, fnup, feup, wn_pad, we_pad, zer):
    faE, faX = fa[:ED], fa[ED:]
    fnupA, fnupB = fnup[:ED], fnup[ED:]
    feupM = feup[:ND]
    feupE = feup[ND:]

    efa, efb = _mm2(ef, faE, feupE)
    xa = _mm1(x, faX)

    mp_p, mr_p = _sc_max(efa, xa, dsti, srci)
    mp8, mr8, ed, er, ea = _prep(mp_p, mr_p, xa)
    S = _sc_zs(efa, ef, mp8, dsti, zer)
    m = _sc_m(ed, er, x, dsti, srci)

    x_new, node_conf = _node_out(S, ea, x, fnupA, fnupB, wn_pad)
    ef_new, edge_conf = _edge_out(m, efb, feupM, we_pad)
    return x_new, ef_new, node_conf, edge_conf


def kernel(x, edge_feats, fa0, fnup0, feup0, fa1, fnup1, feup1, Wn, We, edge_index):
    srci = edge_index[0]
    dsti = edge_index[1]
    wn_pad = jnp.pad(Wn, ((0, 0), (0, 64 - Wn.shape[1])))
    we_pad = jnp.pad(We, ((0, 0), (0, 8 - We.shape[1])))
    zer = jnp.zeros((SR, ND), jnp.float32)

    nf, ef = x, edge_feats
    nf, ef, _, _ = _layer(nf, ef, dsti, srci, fa0, fnup0, feup0,
                          wn_pad, we_pad, zer)
    nf, ef, node_conf, edge_conf = _layer(nf, ef, dsti, srci, fa1, fnup1,
                                          feup1, wn_pad, we_pad, zer)
    return (nf, ef, node_conf[:, :Wn.shape[1]], edge_conf[:, :We.shape[1]])


# speedup vs baseline: 3.2836x; 1.3344x over previous
import functools

import jax
import jax.numpy as jnp
from jax import lax
from jax.experimental import pallas as pl
from jax.experimental.pallas import tpu as pltpu
from jax.experimental.pallas import tpu_sc as plsc

N_NODES = 10000
N_EDGES = 320000
ND = 128
ED = 128
ALPHA = 0.1
BE = 2000
CH = 80
SR = 80

_MESH = plsc.VectorSubcoreMesh(core_axis_name="c", subcore_axis_name="s")


def _leaky(v, a):
    return jnp.where(v > 0, v, a * v)


def _mm2_body(a_ref, w1_ref, w2_ref, o1_ref, o2_ref):
    a = a_ref[...]
    o1_ref[...] = jax.lax.dot(a, w1_ref[...], precision=None)
    o2_ref[...] = jax.lax.dot(a, w2_ref[...], precision=None)


def _mm2(a, w1, w2):
    E, K = a.shape
    return pl.pallas_call(
        _mm2_body,
        grid=(E // BE,),
        in_specs=[
            pl.BlockSpec((BE, K), lambda i: (i, 0)),
            pl.BlockSpec((K, w1.shape[1]), lambda i: (0, 0)),
            pl.BlockSpec((K, w2.shape[1]), lambda i: (0, 0)),
        ],
        out_specs=[
            pl.BlockSpec((BE, w1.shape[1]), lambda i: (i, 0)),
            pl.BlockSpec((BE, w2.shape[1]), lambda i: (i, 0)),
        ],
        out_shape=[
            jax.ShapeDtypeStruct((E, w1.shape[1]), jnp.float32),
            jax.ShapeDtypeStruct((E, w2.shape[1]), jnp.float32),
        ],
    )(a, w1, w2)


def _mm1_body(a_ref, w_ref, o_ref):
    o_ref[...] = jax.lax.dot(a_ref[...], w_ref[...],
                             precision=None)


def _mm1(a, w):
    M, K = a.shape
    BN = 2000
    return pl.pallas_call(
        _mm1_body,
        grid=(M // BN,),
        in_specs=[pl.BlockSpec((BN, K), lambda i: (i, 0)),
                  pl.BlockSpec((K, w.shape[1]), lambda i: (0, 0))],
        out_specs=pl.BlockSpec((BN, w.shape[1]), lambda i: (i, 0)),
        out_shape=jax.ShapeDtypeStruct((M, w.shape[1]), jnp.float32),
    )(a, w)


def _prep_body(mp_ref, mr_ref, xa_ref, mp8_ref, mr8_ref, ed_ref, er_ref, ea_ref):
    mpv = jnp.max(mp_ref[...], axis=0, keepdims=True)
    mrv = jnp.max(mr_ref[...], axis=0, keepdims=True)
    mp8_ref[...] = jnp.broadcast_to(mpv, (8, ND))
    mr8_ref[...] = jnp.broadcast_to(mrv, (8, ND))
    xa = xa_ref[...]
    ed_ref[...] = jnp.exp(xa - mpv)
    er_ref[...] = jnp.exp(xa - mrv)
    ea_ref[...] = jnp.exp(xa)


def _prep(mp_p, mr_p, xa):
    N = xa.shape[0]
    return pl.pallas_call(
        _prep_body,
        grid=(1,),
        in_specs=[
            pl.BlockSpec((32, ND), lambda i: (0, 0)),
            pl.BlockSpec((32, ND), lambda i: (0, 0)),
            pl.BlockSpec((N, ND), lambda i: (0, 0)),
        ],
        out_specs=[
            pl.BlockSpec((8, ND), lambda i: (0, 0)),
            pl.BlockSpec((8, ND), lambda i: (0, 0)),
            pl.BlockSpec((N, ND), lambda i: (0, 0)),
            pl.BlockSpec((N, ND), lambda i: (0, 0)),
            pl.BlockSpec((N, ND), lambda i: (0, 0)),
        ],
        out_shape=[
            jax.ShapeDtypeStruct((8, ND), jnp.float32),
            jax.ShapeDtypeStruct((8, ND), jnp.float32),
            jax.ShapeDtypeStruct((N, ND), jnp.float32),
            jax.ShapeDtypeStruct((N, ND), jnp.float32),
            jax.ShapeDtypeStruct((N, ND), jnp.float32),
        ],
    )(mp_p.reshape(32, ND), mr_p.reshape(32, ND), xa)


def _edge_out_body(m_ref, efb_ref, wm_ref, we_ref, ef_ref, conf_ref):
    hi = None
    raw = jax.lax.dot(m_ref[...], wm_ref[...], precision=hi)
    ef2 = _leaky(_leaky(raw + efb_ref[...], ALPHA), 0.1)
    ef_ref[...] = ef2
    conf_ref[...] = jax.lax.dot(ef2, we_ref[...], precision=hi)


def _edge_out(m, efb, wm, we_pad):
    E = efb.shape[0]
    return pl.pallas_call(
        _edge_out_body,
        grid=(E // BE,),
        in_specs=[
            pl.BlockSpec((BE, ND), lambda i: (i, 0)),
            pl.BlockSpec((BE, ED), lambda i: (i, 0)),
            pl.BlockSpec((ND, ED), lambda i: (0, 0)),
            pl.BlockSpec((ED, we_pad.shape[1]), lambda i: (0, 0)),
        ],
        out_specs=[
            pl.BlockSpec((BE, ED), lambda i: (i, 0)),
            pl.BlockSpec((BE, we_pad.shape[1]), lambda i: (i, 0)),
        ],
        out_shape=[
            jax.ShapeDtypeStruct((E, ED), jnp.float32),
            jax.ShapeDtypeStruct((E, we_pad.shape[1]), jnp.float32),
        ],
    )(m, efb, wm, we_pad)


def _node_out_body(s_ref, ea_ref, x_ref, wa_ref, wb_ref, wn_ref,
                   x_new_ref, conf_ref):
    hi = None
    ea = ea_ref[...]
    z = ea * s_ref[0]
    z2 = ea * s_ref[1] / (z + 1e-05)
    ft = (jax.lax.dot(z2, wa_ref[...], precision=hi)
          + jax.lax.dot(x_ref[...], wb_ref[...], precision=hi))
    xn = _leaky(_leaky(ft, ALPHA), 0.1)
    x_new_ref[...] = xn
    conf_ref[...] = jax.lax.dot(xn, wn_ref[...], precision=hi)


def _node_out(S, ea, x, wa, wb, wn_pad):
    N = x.shape[0]
    BN = 2000
    return pl.pallas_call(
        _node_out_body,
        grid=(N // BN,),
        in_specs=[
            pl.BlockSpec((2, BN, ND), lambda i: (0, i, 0)),
            pl.BlockSpec((BN, ND), lambda i: (i, 0)),
            pl.BlockSpec((BN, ND), lambda i: (i, 0)),
            pl.BlockSpec((ED, ND), lambda i: (0, 0)),
            pl.BlockSpec((ND, ND), lambda i: (0, 0)),
            pl.BlockSpec((ND, wn_pad.shape[1]), lambda i: (0, 0)),
        ],
        out_specs=[
            pl.BlockSpec((BN, ND), lambda i: (i, 0)),
            pl.BlockSpec((BN, wn_pad.shape[1]), lambda i: (i, 0)),
        ],
        out_shape=[
            jax.ShapeDtypeStruct((N, ND), jnp.float32),
            jax.ShapeDtypeStruct((N, wn_pad.shape[1]), jnp.float32),
        ],
    )(S, ea, x, wa, wb, wn_pad)


def _sc_max_body(efa, xa, dsti, srci, mp_out, mr_out,
                 efa_b0, efa_b1, gd_b0, gd_b1, gs_b0, gs_b1,
                 di_b0, di_b1, si_b0, si_b1, mx_b, sem0, sem1):
    c = lax.axis_index("c")
    s = lax.axis_index("s")
    w = c * 16 + s
    e0 = w * (N_EDGES // 32)
    nj = ND // 16
    ninf = jnp.full((16,), -jnp.inf, jnp.float32)

    bufs = ((efa_b0, gd_b0, gs_b0, di_b0, si_b0, sem0),
            (efa_b1, gd_b1, gs_b1, di_b1, si_b1, sem1))

    def fire(k, b):
        efa_b, gd_b, gs_b, di_b, si_b, sem = bufs[b]
        base = pl.multiple_of(e0 + k * CH, 8)
        pltpu.sync_copy(dsti.at[pl.ds(base, CH)], di_b)
        pltpu.sync_copy(srci.at[pl.ds(base, CH)], si_b)
        pltpu.async_copy(efa.at[pl.ds(base, CH)], efa_b, sem)
        pltpu.async_copy(xa.at[di_b], gd_b, sem)
        pltpu.async_copy(xa.at[si_b], gs_b, sem)

    def wait(k, b):
        efa_b, gd_b, gs_b, di_b, si_b, sem = bufs[b]
        base = pl.multiple_of(e0 + k * CH, 8)
        pltpu.make_async_copy(efa.at[pl.ds(base, CH)], efa_b, sem).wait()
        pltpu.make_async_copy(xa.at[di_b], gd_b, sem).wait()
        pltpu.make_async_copy(xa.at[si_b], gs_b, sem).wait()

    def compute(b, acc):
        efa_b, gd_b, gs_b, _, _, _ = bufs[b]

        def row(r, a):
            mp, mr = a
            for j in range(nj):
                jsl = pl.ds(16 * j, 16)
                e = efa_b[r, jsl]
                mp = (mp[:j]
                      + (jnp.maximum(mp[j], e + gd_b[r, jsl]),) + mp[j + 1:])
                mr = (mr[:j]
                      + (jnp.maximum(mr[j], e + gs_b[r, jsl]),) + mr[j + 1:])
            return (mp, mr)

        return lax.fori_loop(0, CH, row, acc)

    nk = N_EDGES // 32 // CH
    fire(0, 0)

    def pair(gg, acc):
        k0 = gg * 2
        fire(k0 + 1, 1)
        wait(k0, 0)
        acc = compute(0, acc)
        fire(k0 + 2, 0)
        wait(k0 + 1, 1)
        return compute(1, acc)

    acc = lax.fori_loop(0, nk // 2, pair, ((ninf,) * nj, (ninf,) * nj))
    wait(nk - 1, 0)
    mp, mr = compute(0, acc)
    for j in range(nj):
        mx_b[0, pl.ds(16 * j, 16)] = mp[j]
        mx_b[1, pl.ds(16 * j, 16)] = mr[j]
    off = pl.multiple_of(w * ND, 8)
    pltpu.sync_copy(mx_b.at[0], mp_out.at[pl.ds(off, ND)])
    pltpu.sync_copy(mx_b.at[1], mr_out.at[pl.ds(off, ND)])


def _sc_max(efa, xa, dsti, srci):
    f = pl.kernel(
        _sc_max_body,
        out_type=[jax.ShapeDtypeStruct((32 * ND,), jnp.float32),
                  jax.ShapeDtypeStruct((32 * ND,), jnp.float32)],
        mesh=_MESH,
        scratch_types=[
            pltpu.VMEM((CH, ND), jnp.float32),
            pltpu.VMEM((CH, ND), jnp.float32),
            pltpu.VMEM((CH, ND), jnp.float32),
            pltpu.VMEM((CH, ND), jnp.float32),
            pltpu.VMEM((CH, ND), jnp.float32),
            pltpu.VMEM((CH, ND), jnp.float32),
            pltpu.VMEM((CH,), jnp.int32),
            pltpu.VMEM((CH,), jnp.int32),
            pltpu.VMEM((CH,), jnp.int32),
            pltpu.VMEM((CH,), jnp.int32),
            pltpu.VMEM((2, ND), jnp.float32),
            pltpu.SemaphoreType.DMA,
            pltpu.SemaphoreType.DMA,
        ],
    )
    return f(efa, xa, dsti, srci)


def _sc_zs_body(efa, ef, mp8, dsti, zer,
                s_out,
                acc_sh,
                efa_b0, efa_b1, ef_b0, ef_b1,
                di_b0, di_b1, mpb, sem0, sem1, sems0, sems1):
    c = lax.axis_index("c")
    s = lax.axis_index("s")
    row0 = s * 640
    nst = jnp.where(s < 15, 640 // SR, 400 // SR)

    def zst(i, _):
        r = pl.multiple_of(row0 + i * SR, 8)
        pltpu.sync_copy(zer, acc_sh.at[pl.ds(r, SR)])
        return 0

    lax.fori_loop(0, nst, zst, 0)
    pltpu.sync_copy(mp8.at[0], mpb)
    plsc.subcore_barrier()

    nj = ND // 16
    mp = tuple(mpb[pl.ds(16 * j, 16)] for j in range(nj))

    e0 = s * (N_EDGES // 16)
    nk = N_EDGES // 16 // CH

    bufs = ((efa_b0, ef_b0, efa_b0, di_b0, sem0, sems0),
            (efa_b1, ef_b1, efa_b1, di_b1, sem1, sems1))

    def fire_lin(k, b):
        efa_b, ef_b, _, di_b, sem, _ = bufs[b]
        base = pl.multiple_of(e0 + k * CH, 8)
        pltpu.async_copy(efa.at[pl.ds(base, CH)], efa_b, sem)
        pltpu.async_copy(dsti.at[pl.ds(base, CH)], di_b, sem)

        @pl.when(c == 1)
        def _():
            pltpu.async_copy(ef.at[pl.ds(base, CH)], ef_b, sem)

    def wait_lin(k, b):
        efa_b, ef_b, _, di_b, sem, _ = bufs[b]
        base = pl.multiple_of(e0 + k * CH, 8)
        pltpu.make_async_copy(efa.at[pl.ds(base, CH)], efa_b, sem).wait()
        pltpu.make_async_copy(dsti.at[pl.ds(base, CH)], di_b, sem).wait()

        @pl.when(c == 1)
        def _():
            pltpu.make_async_copy(ef.at[pl.ds(base, CH)], ef_b, sem).wait()

    def compute(b):
        efa_b, ef_b, w_b, _, _, _ = bufs[b]

        @pl.when(c == 0)
        def _():
            def row(r, _):
                for j in range(nj):
                    jsl = pl.ds(16 * j, 16)
                    w_b[r, jsl] = jnp.exp(efa_b[r, jsl] - mp[j])
                return 0

            lax.fori_loop(0, CH, row, 0)

        @pl.when(c == 1)
        def _():
            def row(r, _):
                for j in range(nj):
                    jsl = pl.ds(16 * j, 16)
                    w_b[r, jsl] = (jnp.exp(efa_b[r, jsl] - mp[j])
                                   * ef_b[r, jsl])
                return 0

            lax.fori_loop(0, CH, row, 0)

    def fire_scatter(b):
        _, _, w_b, di_b, _, sems = bufs[b]
        pltpu.async_copy(w_b, acc_sh.at[di_b], sems, add=True)

    def drain_scatter(b):
        _, _, w_b, di_b, _, sems = bufs[b]
        pltpu.make_async_copy(w_b, acc_sh.at[di_b], sems).wait()

    fire_lin(0, 0)

    def pair(gg, carry):
        k0 = gg * 2

        @pl.when(gg > 0)
        def _():
            drain_scatter(1)

        fire_lin(k0 + 1, 1)
        wait_lin(k0, 0)
        compute(0)
        fire_scatter(0)
        wait_lin(k0 + 1, 1)
        compute(1)
        fire_scatter(1)
        drain_scatter(0)

        @pl.when(gg < (nk // 2 - 1))
        def _():
            fire_lin(k0 + 2, 0)

        return carry

    lax.fori_loop(0, nk // 2, pair, 0)
    drain_scatter(1)
    plsc.subcore_barrier()

    def dmp(i, _):
        r = pl.multiple_of(row0 + i * SR, 8)
        pltpu.sync_copy(acc_sh.at[pl.ds(r, SR)], s_out.at[c, pl.ds(r, SR)])
        return 0

    lax.fori_loop(0, nst, dmp, 0)


def _sc_zs(efa, ef, mp8, dsti, zer):
    f = pl.kernel(
        _sc_zs_body,
        out_type=jax.ShapeDtypeStruct((2, N_NODES, ND), jnp.float32),
        mesh=_MESH,
        scratch_types=[
            pltpu.VMEM_SHARED((N_NODES, ND), jnp.float32),
            pltpu.VMEM((CH, ND), jnp.float32),
            pltpu.VMEM((CH, ND), jnp.float32),
            pltpu.VMEM((CH, ND), jnp.float32),
            pltpu.VMEM((CH, ND), jnp.float32),
            pltpu.VMEM((CH,), jnp.int32),
            pltpu.VMEM((CH,), jnp.int32),
            pltpu.VMEM((ND,), jnp.float32),
            pltpu.SemaphoreType.DMA,
            pltpu.SemaphoreType.DMA,
            pltpu.SemaphoreType.DMA,
            pltpu.SemaphoreType.DMA,
        ],
    )
    return f(efa, ef, mp8, dsti, zer)


def _sc_m_body(ed, er, x, dsti, srci,
               m_out,
               gpd_b0, gpd_b1, grs_b0, grs_b1, xd_b0, xd_b1, xs_b0, xs_b1,
               m_b, di_b0, di_b1, si_b0, si_b1, sem0, sem1):
    c = lax.axis_index("c")
    s = lax.axis_index("s")
    w = c * 16 + s
    e0 = w * (N_EDGES // 32)
    nj = ND // 16
    nk = N_EDGES // 32 // CH

    bufs = ((gpd_b0, grs_b0, xd_b0, xs_b0, di_b0, si_b0, sem0),
            (gpd_b1, grs_b1, xd_b1, xs_b1, di_b1, si_b1, sem1))

    def fire(k, b):
        gpd_b, grs_b, xd_b, xs_b, di_b, si_b, sem = bufs[b]
        base = pl.multiple_of(e0 + k * CH, 8)
        pltpu.sync_copy(dsti.at[pl.ds(base, CH)], di_b)
        pltpu.sync_copy(srci.at[pl.ds(base, CH)], si_b)
        pltpu.async_copy(ed.at[di_b], gpd_b, sem)
        pltpu.async_copy(er.at[si_b], grs_b, sem)
        pltpu.async_copy(x.at[di_b], xd_b, sem)
        pltpu.async_copy(x.at[si_b], xs_b, sem)

    def wait(b):
        gpd_b, grs_b, xd_b, xs_b, di_b, si_b, sem = bufs[b]
        pltpu.make_async_copy(ed.at[di_b], gpd_b, sem).wait()
        pltpu.make_async_copy(er.at[si_b], grs_b, sem).wait()
        pltpu.make_async_copy(x.at[di_b], xd_b, sem).wait()
        pltpu.make_async_copy(x.at[si_b], xs_b, sem).wait()

    def compute_store(k, b):
        gpd_b, grs_b, xd_b, xs_b, _, _, _ = bufs[b]
        base = pl.multiple_of(e0 + k * CH, 8)

        def row(r, _):
            for j in range(nj):
                jsl = pl.ds(16 * j, 16)
                ap = gpd_b[r, jsl]
                ar = grs_b[r, jsl]
                m_b[r, jsl] = ((ar * xs_b[r, jsl] + ap * xd_b[r, jsl])
                               / (ap + ar))
            return 0

        lax.fori_loop(0, CH, row, 0)
        pltpu.sync_copy(m_b, m_out.at[pl.ds(base, CH)])

    fire(0, 0)

    def pair(gg, carry):
        k0 = gg * 2
        fire(k0 + 1, 1)
        wait(0)
        compute_store(k0, 0)
        fire(k0 + 2, 0)
        wait(1)
        compute_store(k0 + 1, 1)
        return carry

    lax.fori_loop(0, nk // 2, pair, 0)
    wait(0)
    compute_store(nk - 1, 0)


def _sc_m(ed, er, x, dsti, srci):
    f = pl.kernel(
        _sc_m_body,
        out_type=jax.ShapeDtypeStruct((N_EDGES, ND), jnp.float32),
        mesh=_MESH,
        scratch_types=[
            pltpu.VMEM((CH, ND), jnp.float32),
            pltpu.VMEM((CH, ND), jnp.float32),
            pltpu.VMEM((CH, ND), jnp.float32),
            pltpu.VMEM((CH, ND), jnp.float32),
            pltpu.VMEM((CH, ND), jnp.float32),
            pltpu.VMEM((CH, ND), jnp.float32),
            pltpu.VMEM((CH, ND), jnp.float32),
            pltpu.VMEM((CH, ND), jnp.float32),
            pltpu.VMEM((CH, ND), jnp.float32),
            pltpu.VMEM((CH,), jnp.int32),
            pltpu.VMEM((CH,), jnp.int32),
            pltpu.VMEM((CH,), jnp.int32),
            pltpu.VMEM((CH,), jnp.int32),
            pltpu.SemaphoreType.DMA,
            pltpu.SemaphoreType.DMA,
        ],
    )
    return f(ed, er, x, dsti, srci)


def _layer(x, ef, dsti, srci, fa, fnup, feup, wn_pad, we_pad, zer):
    faE, faX = fa[:ED], fa[ED:]
    fnupA, fnupB = fnup[:ED], fnup[ED:]
    feupM = feup[:ND]
    feupE = feup[ND:]

    efa, efb = _mm2(ef, faE, feupE)
    xa = _mm1(x, faX)

    mp_p, mr_p = _sc_max(efa, xa, dsti, srci)
    mp8, mr8, ed, er, ea = _prep(mp_p, mr_p, xa)
    S = _sc_zs(efa, ef, mp8, dsti, zer)
    m = _sc_m(ed, er, x, dsti, srci)

    x_new, node_conf = _node_out(S, ea, x, fnupA, fnupB, wn_pad)
    ef_new, edge_conf = _edge_out(m, efb, feupM, we_pad)
    return x_new, ef_new, node_conf, edge_conf


def kernel(x, edge_feats, fa0, fnup0, feup0, fa1, fnup1, feup1, Wn, We, edge_index):
    srci = edge_index[0]
    dsti = edge_index[1]
    wn_pad = jnp.pad(Wn, ((0, 0), (0, 64 - Wn.shape[1])))
    we_pad = jnp.pad(We, ((0, 0), (0, 8 - We.shape[1])))
    zer = jnp.zeros((SR, ND), jnp.float32)

    nf, ef = x, edge_feats
    nf, ef, _, _ = _layer(nf, ef, dsti, srci, fa0, fnup0, feup0,
                          wn_pad, we_pad, zer)
    nf, ef, node_conf, edge_conf = _layer(nf, ef, dsti, srci, fa1, fnup1,
                                          feup1, wn_pad, we_pad, zer)
    return (nf, ef, node_conf[:, :Wn.shape[1]], edge_conf[:, :We.shape[1]])
